# baseline (speedup 1.0000x reference)
import functools

import jax
import jax.numpy as jnp
from jax import lax
from jax.experimental import pallas as pl
from jax.experimental.pallas import tpu as pltpu

_NUM_CLASSES = 81
_ROIS_PER_IMAGE = 128
_FG_ROIS = 32
_BG_ROIS = _ROIS_PER_IMAGE - _FG_ROIS
_FG_THRESH = 0.5
_BG_HI = 0.5
_BG_LO = 0.1
_LANES = 128
_BIGI = 2**30


def _proposal_target_kernel(planes_ref, gt_ref, rois_ref, labels_ref,
                            bbox_ref, inw_ref, fg_ref, bg_ref, asg_ref,
                            colbuf_ref, *, nk, rows, k_gt):
    rw = rows // 8
    f32 = jnp.float32
    i32 = jnp.int32

    x1 = planes_ref[0, 0]
    y1 = planes_ref[0, 1]
    x2 = planes_ref[0, 2]
    y2 = planes_ref[0, 3]
    area_r = (x2 - x1 + 1.0) * (y2 - y1 + 1.0)

    max_ov = jnp.full((rows, _LANES), -1.0, f32)
    assign = jnp.zeros((rows, _LANES), i32)
    for k in range(k_gt):
        gx1 = gt_ref[0, k:k + 1, 0:1]
        gy1 = gt_ref[0, k:k + 1, 1:2]
        gx2 = gt_ref[0, k:k + 1, 2:3]
        gy2 = gt_ref[0, k:k + 1, 3:4]
        ag = (gx2 - gx1 + 1.0) * (gy2 - gy1 + 1.0)
        nz = jnp.where(jnp.abs(gx1) + jnp.abs(gy1) + jnp.abs(gx2) +
                       jnp.abs(gy2) == 0.0, 0.0, 1.0)
        iw = jnp.minimum(x2, gx2) - jnp.maximum(x1, gx1) + 1.0
        ih = jnp.minimum(y2, gy2) - jnp.maximum(y1, gy1) + 1.0
        inter = jnp.maximum(iw, 0.0) * jnp.maximum(ih, 0.0)
        iou = (inter / (area_r + (ag - inter))) * nz
        better = iou > max_ov
        assign = jnp.where(better, k, assign)
        max_ov = jnp.maximum(max_ov, iou)

    ridx = lax.broadcasted_iota(i32, (rows, _LANES), 0)
    cidx = lax.broadcasted_iota(i32, (rows, _LANES), 1)
    valid = (ridx * _LANES + cidx) < nk
    fg_s = jnp.where(valid, jnp.where(max_ov >= _FG_THRESH, max_ov, -1.0), -2.0)
    bg_s = jnp.where(
        valid,
        jnp.where((max_ov < _BG_HI) & (max_ov >= _BG_LO), max_ov, -1.0), -2.0)
    fg_ref[...] = fg_s
    bg_ref[...] = bg_s
    asg_ref[...] = assign

    rmap = (lax.broadcasted_iota(i32, (8, rw), 0) * rw +
            lax.broadcasted_iota(i32, (8, rw), 1))
    lane_i = lax.broadcasted_iota(i32, (1, _LANES), 1)
    rm_fg = jnp.max(fg_s.reshape(8, rw, _LANES), axis=2)
    rm_bg = jnp.max(bg_s.reshape(8, rw, _LANES), axis=2)

    def make_body(score_ref, is_fg, base):
        def body(it, carry):
            rm, lab_acc = carry
            v = jnp.max(rm)
            r = jnp.min(jnp.where(rm == v, rmap, _BIGI))
            row = score_ref[pl.ds(r, 1), :]
            c = jnp.min(jnp.where(row == v, lane_i, _BIGI))
            lane_sel = lane_i == c
            nrow = jnp.where(lane_sel, -3.0, row)
            score_ref[pl.ds(r, 1), :] = nrow
            rm = jnp.where(rmap == r, jnp.max(nrow), rm)
            slot = base + it
            for j in range(4):
                prow = planes_ref[0, j, pl.ds(r, 1), :]
                v11 = jnp.sum(jnp.where(lane_sel, prow, 0.0), axis=1,
                              keepdims=True)
                colbuf_ref[pl.ds(slot, 1), j:j + 1] = v11
            arow = asg_ref[pl.ds(r, 1), :]
            ga = jnp.sum(jnp.where(lane_sel, arow, 0))
            grow = gt_ref[0, pl.ds(ga, 1), :]
            for j in range(4):
                colbuf_ref[pl.ds(slot, 1), 4 + j:5 + j] = grow[:, j:j + 1]
            if is_fg:
                labv = jnp.where(v >= _FG_THRESH, grow[:, 4:5],
                                 jnp.zeros((1, 1), f32))
                lab_acc = jnp.where(lane_i == slot, labv, lab_acc)
            else:
                labv = jnp.zeros((1, 1), f32)
            colbuf_ref[pl.ds(slot, 1), 8:9] = labv
            return rm, lab_acc
        return body

    lab0 = jnp.zeros((1, _LANES), f32)
    rm_fg, lab_acc = lax.fori_loop(0, _FG_ROIS, make_body(fg_ref, True, 0),
                                   (rm_fg, lab0))
    rm_bg, lab_acc = lax.fori_loop(0, _BG_ROIS, make_body(bg_ref, False,
                                                          _FG_ROIS),
                                   (rm_bg, lab_acc))

    cb = colbuf_ref[...]
    ex1 = cb[:, 0:1]
    ey1 = cb[:, 1:2]
    ex2 = cb[:, 2:3]
    ey2 = cb[:, 3:4]
    gx1 = cb[:, 4:5]
    gy1 = cb[:, 5:6]
    gx2 = cb[:, 6:7]
    gy2 = cb[:, 7:8]
    cls = cb[:, 8:9]

    ew = ex2 - ex1 + 1.0
    eh = ey2 - ey1 + 1.0
    ecx = ex1 + 0.5 * ew
    ecy = ey1 + 0.5 * eh
    gw = gx2 - gx1 + 1.0
    gh = gy2 - gy1 + 1.0
    gcx = gx1 + 0.5 * gw
    gcy = gy1 + 0.5 * gh
    tx = (gcx - ecx) / ew * 10.0
    ty = (gcy - ecy) / eh * 10.0
    tw = jnp.log(gw / ew) * 5.0
    th = jnp.log(gh / eh) * 5.0

    bcol = jnp.full((_ROIS_PER_IMAGE, 1), pl.program_id(0), f32)
    rois_ref[0] = jnp.concatenate([bcol, ex1, ey1, ex2, ey2], axis=1)
    labels_ref[0] = lab_acc

    clsi = cls.astype(i32)
    maskc = cls > 0.0
    jj = lax.broadcasted_iota(i32, (_ROIS_PER_IMAGE, 4 * _NUM_CLASSES), 1)
    cj = jj >> 2
    dj = jj & 3
    sel = (cj == clsi) & maskc
    tval = jnp.where(dj == 0, tx,
                     jnp.where(dj == 1, ty, jnp.where(dj == 2, tw, th)))
    bbox_ref[0] = jnp.where(sel, tval, 0.0)
    inw_ref[0] = jnp.where(sel, 1.0, 0.0)


@jax.jit
def kernel(all_rois, gt_boxes, num_boxes):
    del num_boxes
    B, N, _ = all_rois.shape
    K = gt_boxes.shape[1]
    nk = N + K
    rows = -(-nk // _LANES)
    rows = -(-rows // 8) * 8
    p = rows * _LANES

    coords = jnp.concatenate([all_rois[:, :, 1:5], gt_boxes[:, :, :4]], axis=1)
    coords = jnp.pad(coords, ((0, 0), (0, p - nk), (0, 0)))
    planes = coords.transpose(0, 2, 1).reshape(B, 4, rows, _LANES)

    kern = functools.partial(_proposal_target_kernel, nk=nk, rows=rows, k_gt=K)
    out_shapes = (
        jax.ShapeDtypeStruct((B, _ROIS_PER_IMAGE, 5), jnp.float32),
        jax.ShapeDtypeStruct((B, 1, _ROIS_PER_IMAGE), jnp.float32),
        jax.ShapeDtypeStruct((B, _ROIS_PER_IMAGE, 4 * _NUM_CLASSES),
                             jnp.float32),
        jax.ShapeDtypeStruct((B, _ROIS_PER_IMAGE, 4 * _NUM_CLASSES),
                             jnp.float32),
    )
    rois, labels3, bbox, inw = pl.pallas_call(
        kern,
        grid=(B,),
        in_specs=[
            pl.BlockSpec((1, 4, rows, _LANES), lambda b: (b, 0, 0, 0)),
            pl.BlockSpec((1, K, 5), lambda b: (b, 0, 0)),
        ],
        out_specs=[
            pl.BlockSpec((1, _ROIS_PER_IMAGE, 5), lambda b: (b, 0, 0)),
            pl.BlockSpec((1, 1, _ROIS_PER_IMAGE), lambda b: (b, 0, 0)),
            pl.BlockSpec((1, _ROIS_PER_IMAGE, 4 * _NUM_CLASSES),
                         lambda b: (b, 0, 0)),
            pl.BlockSpec((1, _ROIS_PER_IMAGE, 4 * _NUM_CLASSES),
                         lambda b: (b, 0, 0)),
        ],
        out_shape=out_shapes,
        scratch_shapes=[
            pltpu.VMEM((rows, _LANES), jnp.float32),
            pltpu.VMEM((rows, _LANES), jnp.float32),
            pltpu.VMEM((rows, _LANES), jnp.int32),
            pltpu.VMEM((_ROIS_PER_IMAGE, 16), jnp.float32),
        ],
        compiler_params=pltpu.CompilerParams(
            dimension_semantics=("parallel",)),
    )(planes, gt_boxes)
    return rois, labels3.reshape(B, _ROIS_PER_IMAGE), bbox, inw


# speedup vs baseline: 1.0001x; 1.0001x over previous
import functools

import jax
import jax.numpy as jnp
from jax import lax
from jax.experimental import pallas as pl
from jax.experimental.pallas import tpu as pltpu

_NUM_CLASSES = 81
_ROIS_PER_IMAGE = 128
_FG_ROIS = 32
_BG_ROIS = _ROIS_PER_IMAGE - _FG_ROIS
_FG_THRESH = 0.5
_BG_HI = 0.5
_BG_LO = 0.1
_LANES = 128
_BIGI = 2**30


def _proposal_target_kernel(planes_ref, gt_ref, rois_ref, labels_ref,
                            bbox_ref, inw_ref, fg_ref, bg_ref, asg_ref,
                            colbuf_ref, *, nk, rows, k_gt):
    rw = rows // 8
    f32 = jnp.float32
    i32 = jnp.int32

    x1 = planes_ref[0, 0]
    y1 = planes_ref[0, 1]
    x2 = planes_ref[0, 2]
    y2 = planes_ref[0, 3]
    area_r = (x2 - x1 + 1.0) * (y2 - y1 + 1.0)

    max_ov = jnp.full((rows, _LANES), -1.0, f32)
    assign = jnp.zeros((rows, _LANES), i32)
    for k in range(k_gt):
        gx1 = gt_ref[0, k:k + 1, 0:1]
        gy1 = gt_ref[0, k:k + 1, 1:2]
        gx2 = gt_ref[0, k:k + 1, 2:3]
        gy2 = gt_ref[0, k:k + 1, 3:4]
        ag = (gx2 - gx1 + 1.0) * (gy2 - gy1 + 1.0)
        nz = jnp.where(jnp.abs(gx1) + jnp.abs(gy1) + jnp.abs(gx2) +
                       jnp.abs(gy2) == 0.0, 0.0, 1.0)
        iw = jnp.minimum(x2, gx2) - jnp.maximum(x1, gx1) + 1.0
        ih = jnp.minimum(y2, gy2) - jnp.maximum(y1, gy1) + 1.0
        inter = jnp.maximum(iw, 0.0) * jnp.maximum(ih, 0.0)
        iou = (inter / ((area_r + ag) - inter)) * nz
        better = iou > max_ov
        assign = jnp.where(better, k, assign)
        max_ov = jnp.maximum(max_ov, iou)

    ridx = lax.broadcasted_iota(i32, (rows, _LANES), 0)
    cidx = lax.broadcasted_iota(i32, (rows, _LANES), 1)
    valid = (ridx * _LANES + cidx) < nk
    fg_s = jnp.where(valid, jnp.where(max_ov >= _FG_THRESH, max_ov, -1.0), -2.0)
    bg_s = jnp.where(
        valid,
        jnp.where((max_ov < _BG_HI) & (max_ov >= _BG_LO), max_ov, -1.0), -2.0)
    fg_ref[...] = fg_s
    bg_ref[...] = bg_s
    asg_ref[...] = assign

    rmap = (lax.broadcasted_iota(i32, (8, rw), 0) * rw +
            lax.broadcasted_iota(i32, (8, rw), 1))
    lane_i = lax.broadcasted_iota(i32, (1, _LANES), 1)
    rm_fg = jnp.max(fg_s.reshape(8, rw, _LANES), axis=2)
    rm_bg = jnp.max(bg_s.reshape(8, rw, _LANES), axis=2)

    def make_body(score_ref, is_fg, base):
        def body(it, carry):
            rm, lab_acc = carry
            v = jnp.max(rm)
            r = jnp.min(jnp.where(rm == v, rmap, _BIGI))
            row = score_ref[pl.ds(r, 1), :]
            c = jnp.min(jnp.where(row == v, lane_i, _BIGI))
            lane_sel = lane_i == c
            nrow = jnp.where(lane_sel, -3.0, row)
            score_ref[pl.ds(r, 1), :] = nrow
            rm = jnp.where(rmap == r, jnp.max(nrow), rm)
            slot = base + it
            for j in range(4):
                prow = planes_ref[0, j, pl.ds(r, 1), :]
                v11 = jnp.sum(jnp.where(lane_sel, prow, 0.0), axis=1,
                              keepdims=True)
                colbuf_ref[pl.ds(slot, 1), j:j + 1] = v11
            arow = asg_ref[pl.ds(r, 1), :]
            ga = jnp.sum(jnp.where(lane_sel, arow, 0))
            grow = gt_ref[0, pl.ds(ga, 1), :]
            for j in range(4):
                colbuf_ref[pl.ds(slot, 1), 4 + j:5 + j] = grow[:, j:j + 1]
            if is_fg:
                labv = jnp.where(v >= _FG_THRESH, grow[:, 4:5],
                                 jnp.zeros((1, 1), f32))
                lab_acc = jnp.where(lane_i == slot, labv, lab_acc)
            else:
                labv = jnp.zeros((1, 1), f32)
            colbuf_ref[pl.ds(slot, 1), 8:9] = labv
            return rm, lab_acc
        return body

    lab0 = jnp.zeros((1, _LANES), f32)
    rm_fg, lab_acc = lax.fori_loop(0, _FG_ROIS, make_body(fg_ref, True, 0),
                                   (rm_fg, lab0))
    rm_bg, lab_acc = lax.fori_loop(0, _BG_ROIS, make_body(bg_ref, False,
                                                          _FG_ROIS),
                                   (rm_bg, lab_acc))

    cb = colbuf_ref[...]
    ex1 = cb[:, 0:1]
    ey1 = cb[:, 1:2]
    ex2 = cb[:, 2:3]
    ey2 = cb[:, 3:4]
    gx1 = cb[:, 4:5]
    gy1 = cb[:, 5:6]
    gx2 = cb[:, 6:7]
    gy2 = cb[:, 7:8]
    cls = cb[:, 8:9]

    ew = ex2 - ex1 + 1.0
    eh = ey2 - ey1 + 1.0
    ecx = ex1 + 0.5 * ew
    ecy = ey1 + 0.5 * eh
    gw = gx2 - gx1 + 1.0
    gh = gy2 - gy1 + 1.0
    gcx = gx1 + 0.5 * gw
    gcy = gy1 + 0.5 * gh
    tx = ((gcx - ecx) / ew) / 0.1
    ty = ((gcy - ecy) / eh) / 0.1
    tw = jnp.log(gw / ew) / 0.2
    th = jnp.log(gh / eh) / 0.2

    bcol = jnp.full((_ROIS_PER_IMAGE, 1), pl.program_id(0), f32)
    rois_ref[0] = jnp.concatenate([bcol, ex1, ey1, ex2, ey2], axis=1)
    labels_ref[0] = lab_acc

    clsi = cls.astype(i32)
    maskc = cls > 0.0
    jj = lax.broadcasted_iota(i32, (_ROIS_PER_IMAGE, 4 * _NUM_CLASSES), 1)
    cj = jj >> 2
    dj = jj & 3
    sel = (cj == clsi) & maskc
    tval = jnp.where(dj == 0, tx,
                     jnp.where(dj == 1, ty, jnp.where(dj == 2, tw, th)))
    bbox_ref[0] = jnp.where(sel, tval, 0.0)
    inw_ref[0] = jnp.where(sel, 1.0, 0.0)


@jax.jit
def kernel(all_rois, gt_boxes, num_boxes):
    del num_boxes
    B, N, _ = all_rois.shape
    K = gt_boxes.shape[1]
    nk = N + K
    rows = -(-nk // _LANES)
    rows = -(-rows // 8) * 8
    p = rows * _LANES

    coords = jnp.concatenate([all_rois[:, :, 1:5], gt_boxes[:, :, :4]], axis=1)
    coords = jnp.pad(coords, ((0, 0), (0, p - nk), (0, 0)))
    planes = coords.transpose(0, 2, 1).reshape(B, 4, rows, _LANES)

    kern = functools.partial(_proposal_target_kernel, nk=nk, rows=rows, k_gt=K)
    out_shapes = (
        jax.ShapeDtypeStruct((B, _ROIS_PER_IMAGE, 5), jnp.float32),
        jax.ShapeDtypeStruct((B, 1, _ROIS_PER_IMAGE), jnp.float32),
        jax.ShapeDtypeStruct((B, _ROIS_PER_IMAGE, 4 * _NUM_CLASSES),
                             jnp.float32),
        jax.ShapeDtypeStruct((B, _ROIS_PER_IMAGE, 4 * _NUM_CLASSES),
                             jnp.float32),
    )
    rois, labels3, bbox, inw = pl.pallas_call(
        kern,
        grid=(B,),
        in_specs=[
            pl.BlockSpec((1, 4, rows, _LANES), lambda b: (b, 0, 0, 0)),
            pl.BlockSpec((1, K, 5), lambda b: (b, 0, 0)),
        ],
        out_specs=[
            pl.BlockSpec((1, _ROIS_PER_IMAGE, 5), lambda b: (b, 0, 0)),
            pl.BlockSpec((1, 1, _ROIS_PER_IMAGE), lambda b: (b, 0, 0)),
            pl.BlockSpec((1, _ROIS_PER_IMAGE, 4 * _NUM_CLASSES),
                         lambda b: (b, 0, 0)),
            pl.BlockSpec((1, _ROIS_PER_IMAGE, 4 * _NUM_CLASSES),
                         lambda b: (b, 0, 0)),
        ],
        out_shape=out_shapes,
        scratch_shapes=[
            pltpu.VMEM((rows, _LANES), jnp.float32),
            pltpu.VMEM((rows, _LANES), jnp.float32),
            pltpu.VMEM((rows, _LANES), jnp.int32),
            pltpu.VMEM((_ROIS_PER_IMAGE, 16), jnp.float32),
        ],
        compiler_params=pltpu.CompilerParams(
            dimension_semantics=("parallel",)),
    )(planes, gt_boxes)
    return rois, labels3.reshape(B, _ROIS_PER_IMAGE), bbox, inw


# speedup vs baseline: 1.2557x; 1.2556x over previous
import functools

import jax
import jax.numpy as jnp
from jax import lax
from jax.experimental import pallas as pl
from jax.experimental.pallas import tpu as pltpu

_NUM_CLASSES = 81
_ROIS_PER_IMAGE = 128
_FG_ROIS = 32
_BG_ROIS = _ROIS_PER_IMAGE - _FG_ROIS
_FG_THRESH = 0.5
_BG_HI = 0.5
_BG_LO = 0.1
_LANES = 128
_BIGI = 2**30


def _stage1_kernel(planes_ref, gt_ref, fg_ref, bg_ref, asg_ref, *,
                   nk, rows, k_gt):
    f32 = jnp.float32
    i32 = jnp.int32

    x1 = planes_ref[0, 0]
    y1 = planes_ref[0, 1]
    x2 = planes_ref[0, 2]
    y2 = planes_ref[0, 3]
    area_r = (x2 - x1 + 1.0) * (y2 - y1 + 1.0)

    max_ov = jnp.full((rows, _LANES), -1.0, f32)
    assign = jnp.zeros((rows, _LANES), i32)
    for k in range(k_gt):
        gx1 = gt_ref[0, k:k + 1, 0:1]
        gy1 = gt_ref[0, k:k + 1, 1:2]
        gx2 = gt_ref[0, k:k + 1, 2:3]
        gy2 = gt_ref[0, k:k + 1, 3:4]
        ag = (gx2 - gx1 + 1.0) * (gy2 - gy1 + 1.0)
        nz = jnp.where(jnp.abs(gx1) + jnp.abs(gy1) + jnp.abs(gx2) +
                       jnp.abs(gy2) == 0.0, 0.0, 1.0)
        iw = jnp.minimum(x2, gx2) - jnp.maximum(x1, gx1) + 1.0
        ih = jnp.minimum(y2, gy2) - jnp.maximum(y1, gy1) + 1.0
        inter = jnp.maximum(iw, 0.0) * jnp.maximum(ih, 0.0)
        iou = (inter / ((area_r + ag) - inter)) * nz
        better = iou > max_ov
        assign = jnp.where(better, k, assign)
        max_ov = jnp.maximum(max_ov, iou)

    ridx = lax.broadcasted_iota(i32, (rows, _LANES), 0)
    cidx = lax.broadcasted_iota(i32, (rows, _LANES), 1)
    valid = (ridx * _LANES + cidx) < nk
    fg_ref[0] = jnp.where(
        valid, jnp.where(max_ov >= _FG_THRESH, max_ov, -1.0), -2.0)
    bg_ref[0] = jnp.where(
        valid,
        jnp.where((max_ov < _BG_HI) & (max_ov >= _BG_LO), max_ov, -1.0), -2.0)
    asg_ref[0] = assign


def _select_kernel(planes_ref, gt_ref, fgin_ref, bgin_ref, asg_ref,
                   rois_ref, labels_ref, bbox_ref, inw_ref,
                   fg_ref, bg_ref, colbuf_ref, *, nb, rows):
    f32 = jnp.float32
    i32 = jnp.int32
    rw = rows // 8

    for b in range(nb):
        fg_ref[b * rows:(b + 1) * rows, :] = fgin_ref[b]
        bg_ref[b * rows:(b + 1) * rows, :] = bgin_ref[b]

    rmap = (lax.broadcasted_iota(i32, (8, rw), 0) * rw +
            lax.broadcasted_iota(i32, (8, rw), 1))
    lane_i = lax.broadcasted_iota(i32, (1, _LANES), 1)
    su_i = lax.broadcasted_iota(i32, (nb, _LANES), 0)
    la_i = lax.broadcasted_iota(i32, (nb, _LANES), 1)

    rm_fg = jnp.concatenate(
        [jnp.max(fgin_ref[b].reshape(8, rw, _LANES), axis=2)
         for b in range(nb)], axis=0)
    rm_bg = jnp.concatenate(
        [jnp.max(bgin_ref[b].reshape(8, rw, _LANES), axis=2)
         for b in range(nb)], axis=0)

    def step(b, rm_b, score_ref, slot, is_fg, lab_acc):
        v = jnp.max(rm_b)
        r = jnp.min(jnp.where(rm_b == v, rmap, _BIGI))
        gr = b * rows + r
        row = score_ref[pl.ds(gr, 1), :]
        c = jnp.min(jnp.where(row == v, lane_i, _BIGI))
        lane_sel = lane_i == c
        nrow = jnp.where(lane_sel, -3.0, row)
        score_ref[pl.ds(gr, 1), :] = nrow
        rm_b = jnp.where(rmap == r, jnp.max(nrow), rm_b)
        crow = b * _ROIS_PER_IMAGE + slot
        for j in range(4):
            prow = planes_ref[b, j, pl.ds(r, 1), :]
            v11 = jnp.sum(jnp.where(lane_sel, prow, 0.0), axis=1,
                          keepdims=True)
            colbuf_ref[pl.ds(crow, 1), j:j + 1] = v11
        arow = asg_ref[b, pl.ds(r, 1), :]
        ga = jnp.sum(jnp.where(lane_sel, arow, 0))
        grow = gt_ref[b, pl.ds(ga, 1), :]
        for j in range(4):
            colbuf_ref[pl.ds(crow, 1), 4 + j:5 + j] = grow[:, j:j + 1]
        if is_fg:
            labv = jnp.where(v >= _FG_THRESH, grow[:, 4:5],
                             jnp.zeros((1, 1), f32))
            lab_acc = jnp.where((su_i == b) & (la_i == slot), labv, lab_acc)
        else:
            labv = jnp.zeros((1, 1), f32)
        colbuf_ref[pl.ds(crow, 1), 8:9] = labv
        return rm_b, lab_acc

    def body_fgbg(it, carry):
        rm_fg, rm_bg, lab_acc = carry
        rmf_parts, rmb_parts = [], []
        for b in range(nb):
            rm_b = rm_fg[b * 8:(b + 1) * 8, :]
            rm_b, lab_acc = step(b, rm_b, fg_ref, it, True, lab_acc)
            rmf_parts.append(rm_b)
            rm_b2 = rm_bg[b * 8:(b + 1) * 8, :]
            rm_b2, _ = step(b, rm_b2, bg_ref, _FG_ROIS + it, False, lab_acc)
            rmb_parts.append(rm_b2)
        return (jnp.concatenate(rmf_parts, axis=0),
                jnp.concatenate(rmb_parts, axis=0), lab_acc)

    def body_bg(it, carry):
        rm_bg, lab_acc = carry
        rmb_parts = []
        for b in range(nb):
            rm_b = rm_bg[b * 8:(b + 1) * 8, :]
            rm_b, _ = step(b, rm_b, bg_ref, _FG_ROIS + it, False, lab_acc)
            rmb_parts.append(rm_b)
        return jnp.concatenate(rmb_parts, axis=0), lab_acc

    lab0 = jnp.zeros((nb, _LANES), f32)
    rm_fg, rm_bg, lab_acc = lax.fori_loop(
        0, _FG_ROIS, body_fgbg, (rm_fg, rm_bg, lab0))
    rm_bg, lab_acc = lax.fori_loop(
        _FG_ROIS, _BG_ROIS, body_bg, (rm_bg, lab_acc))

    labels_ref[...] = lab_acc
    jj = lax.broadcasted_iota(i32, (_ROIS_PER_IMAGE, 4 * _NUM_CLASSES), 1)
    cj = jj >> 2
    dj = jj & 3
    for b in range(nb):
        cb = colbuf_ref[b * _ROIS_PER_IMAGE:(b + 1) * _ROIS_PER_IMAGE, :]
        ex1 = cb[:, 0:1]
        ey1 = cb[:, 1:2]
        ex2 = cb[:, 2:3]
        ey2 = cb[:, 3:4]
        gx1 = cb[:, 4:5]
        gy1 = cb[:, 5:6]
        gx2 = cb[:, 6:7]
        gy2 = cb[:, 7:8]
        cls = cb[:, 8:9]

        ew = ex2 - ex1 + 1.0
        eh = ey2 - ey1 + 1.0
        ecx = ex1 + 0.5 * ew
        ecy = ey1 + 0.5 * eh
        gw = gx2 - gx1 + 1.0
        gh = gy2 - gy1 + 1.0
        gcx = gx1 + 0.5 * gw
        gcy = gy1 + 0.5 * gh
        tx = ((gcx - ecx) / ew) / 0.1
        ty = ((gcy - ecy) / eh) / 0.1
        tw = jnp.log(gw / ew) / 0.2
        th = jnp.log(gh / eh) / 0.2

        bcol = jnp.full((_ROIS_PER_IMAGE, 1), float(b), f32)
        rois_ref[b] = jnp.concatenate([bcol, ex1, ey1, ex2, ey2], axis=1)
        clsi = cls.astype(i32)
        maskc = cls > 0.0
        sel = (cj == clsi) & maskc
        tval = jnp.where(dj == 0, tx,
                         jnp.where(dj == 1, ty, jnp.where(dj == 2, tw, th)))
        bbox_ref[b] = jnp.where(sel, tval, 0.0)
        inw_ref[b] = jnp.where(sel, 1.0, 0.0)


@jax.jit
def kernel(all_rois, gt_boxes, num_boxes):
    del num_boxes
    B, N, _ = all_rois.shape
    K = gt_boxes.shape[1]
    nk = N + K
    rows = -(-nk // _LANES)
    rows = -(-rows // 8) * 8
    p = rows * _LANES

    coords = jnp.concatenate([all_rois[:, :, 1:5], gt_boxes[:, :, :4]], axis=1)
    coords = jnp.pad(coords, ((0, 0), (0, p - nk), (0, 0)))
    planes = coords.transpose(0, 2, 1).reshape(B, 4, rows, _LANES)

    s1 = functools.partial(_stage1_kernel, nk=nk, rows=rows, k_gt=K)
    fg_s, bg_s, asg = pl.pallas_call(
        s1,
        grid=(B,),
        in_specs=[
            pl.BlockSpec((1, 4, rows, _LANES), lambda b: (b, 0, 0, 0)),
            pl.BlockSpec((1, K, 5), lambda b: (b, 0, 0)),
        ],
        out_specs=[
            pl.BlockSpec((1, rows, _LANES), lambda b: (b, 0, 0)),
            pl.BlockSpec((1, rows, _LANES), lambda b: (b, 0, 0)),
            pl.BlockSpec((1, rows, _LANES), lambda b: (b, 0, 0)),
        ],
        out_shape=(
            jax.ShapeDtypeStruct((B, rows, _LANES), jnp.float32),
            jax.ShapeDtypeStruct((B, rows, _LANES), jnp.float32),
            jax.ShapeDtypeStruct((B, rows, _LANES), jnp.int32),
        ),
    )(planes, gt_boxes)

    sel = functools.partial(_select_kernel, nb=B, rows=rows)
    full = lambda *shape: pl.BlockSpec(shape, lambda: tuple(0 for _ in shape))
    rois, labels, bbox, inw = pl.pallas_call(
        sel,
        grid=(),
        in_specs=[
            full(B, 4, rows, _LANES),
            full(B, K, 5),
            full(B, rows, _LANES),
            full(B, rows, _LANES),
            full(B, rows, _LANES),
        ],
        out_specs=[
            full(B, _ROIS_PER_IMAGE, 5),
            full(B, _ROIS_PER_IMAGE),
            full(B, _ROIS_PER_IMAGE, 4 * _NUM_CLASSES),
            full(B, _ROIS_PER_IMAGE, 4 * _NUM_CLASSES),
        ],
        out_shape=(
            jax.ShapeDtypeStruct((B, _ROIS_PER_IMAGE, 5), jnp.float32),
            jax.ShapeDtypeStruct((B, _ROIS_PER_IMAGE), jnp.float32),
            jax.ShapeDtypeStruct((B, _ROIS_PER_IMAGE, 4 * _NUM_CLASSES),
                                 jnp.float32),
            jax.ShapeDtypeStruct((B, _ROIS_PER_IMAGE, 4 * _NUM_CLASSES),
                                 jnp.float32),
        ),
        scratch_shapes=[
            pltpu.VMEM((B * rows, _LANES), jnp.float32),
            pltpu.VMEM((B * rows, _LANES), jnp.float32),
            pltpu.VMEM((B * _ROIS_PER_IMAGE, 16), jnp.float32),
        ],
    )(planes, gt_boxes, fg_s, bg_s, asg)
    return rois, labels, bbox, inw


# speedup vs baseline: 2.5691x; 2.0458x over previous
import functools

import jax
import jax.numpy as jnp
from jax import lax
from jax.experimental import pallas as pl
from jax.experimental.pallas import tpu as pltpu

_NUM_CLASSES = 81
_ROIS_PER_IMAGE = 128
_FG_ROIS = 32
_BG_ROIS = _ROIS_PER_IMAGE - _FG_ROIS
_FG_THRESH = 0.5
_BG_HI = 0.5
_BG_LO = 0.1
_LANES = 128
_BIGI = 2**30


def _stage1_kernel(planes_ref, gt_ref, fg_ref, bg_ref, asg_ref, *,
                   nk, rows, k_gt):
    f32 = jnp.float32
    i32 = jnp.int32

    x1 = planes_ref[0, 0]
    y1 = planes_ref[0, 1]
    x2 = planes_ref[0, 2]
    y2 = planes_ref[0, 3]
    area_r = (x2 - x1 + 1.0) * (y2 - y1 + 1.0)

    max_ov = jnp.full((rows, _LANES), -1.0, f32)
    assign = jnp.zeros((rows, _LANES), i32)
    for k in range(k_gt):
        gx1 = gt_ref[0, k:k + 1, 0:1]
        gy1 = gt_ref[0, k:k + 1, 1:2]
        gx2 = gt_ref[0, k:k + 1, 2:3]
        gy2 = gt_ref[0, k:k + 1, 3:4]
        ag = (gx2 - gx1 + 1.0) * (gy2 - gy1 + 1.0)
        nz = jnp.where(jnp.abs(gx1) + jnp.abs(gy1) + jnp.abs(gx2) +
                       jnp.abs(gy2) == 0.0, 0.0, 1.0)
        iw = jnp.minimum(x2, gx2) - jnp.maximum(x1, gx1) + 1.0
        ih = jnp.minimum(y2, gy2) - jnp.maximum(y1, gy1) + 1.0
        inter = jnp.maximum(iw, 0.0) * jnp.maximum(ih, 0.0)
        iou = (inter / ((area_r + ag) - inter)) * nz
        better = iou > max_ov
        assign = jnp.where(better, k, assign)
        max_ov = jnp.maximum(max_ov, iou)

    ridx = lax.broadcasted_iota(i32, (rows, _LANES), 0)
    cidx = lax.broadcasted_iota(i32, (rows, _LANES), 1)
    valid = (ridx * _LANES + cidx) < nk
    fg_ref[0] = jnp.where(
        valid, jnp.where(max_ov >= _FG_THRESH, max_ov, -1.0), -2.0)
    bg_ref[0] = jnp.where(
        valid,
        jnp.where((max_ov < _BG_HI) & (max_ov >= _BG_LO), max_ov, -1.0), -2.0)
    asg_ref[0] = assign


def _select_kernel(planes_ref, gt_ref, fgin_ref, bgin_ref, asg_ref,
                   rois_ref, labels_ref, bbox_ref, inw_ref,
                   fg_ref, bg_ref, colbuf_ref, *, nb, rows):
    f32 = jnp.float32
    i32 = jnp.int32
    rw = rows // 8

    for b in range(nb):
        fg_ref[b * rows:(b + 1) * rows, :] = fgin_ref[b]
        bg_ref[b * rows:(b + 1) * rows, :] = bgin_ref[b]

    rmap = (lax.broadcasted_iota(i32, (8, rw), 0) * rw +
            lax.broadcasted_iota(i32, (8, rw), 1))
    lane_i = lax.broadcasted_iota(i32, (1, _LANES), 1)
    su_i = lax.broadcasted_iota(i32, (nb, _LANES), 0)
    la_i = lax.broadcasted_iota(i32, (nb, _LANES), 1)

    rm_fg = jnp.concatenate(
        [jnp.max(fgin_ref[b].reshape(8, rw, _LANES), axis=2)
         for b in range(nb)], axis=0)
    rm_bg = jnp.concatenate(
        [jnp.max(bgin_ref[b].reshape(8, rw, _LANES), axis=2)
         for b in range(nb)], axis=0)

    kio = lax.broadcasted_iota(jnp.int32, (gt_ref.shape[1], 1), 0)

    def step(b, rm_b, score_ref, slot, is_fg, lab_acc):
        v = jnp.max(rm_b, axis=(0, 1), keepdims=True)
        r = jnp.min(jnp.where(rm_b == v, rmap, _BIGI))
        gr = b * rows + r
        row = score_ref[pl.ds(gr, 1), :]
        c11 = jnp.min(jnp.where(row == v, lane_i, _BIGI), axis=1,
                      keepdims=True)
        lane_sel = lane_i == c11
        nrow = jnp.where(lane_sel, -3.0, row)
        score_ref[pl.ds(gr, 1), :] = nrow
        rm_b = jnp.where(rmap == r, jnp.max(nrow, axis=1, keepdims=True),
                         rm_b)
        crow = b * _ROIS_PER_IMAGE + slot
        for j in range(4):
            prow = planes_ref[b, j, pl.ds(r, 1), :]
            v11 = jnp.sum(jnp.where(lane_sel, prow, 0.0), axis=1,
                          keepdims=True)
            colbuf_ref[pl.ds(crow, 1), j:j + 1] = v11
        arow = asg_ref[b, pl.ds(r, 1), :]
        ga11 = jnp.sum(jnp.where(lane_sel, arow, 0), axis=1,
                       keepdims=True)
        gt_all = gt_ref[b]
        grow = jnp.sum(jnp.where(kio == ga11, gt_all, 0.0), axis=0,
                       keepdims=True)
        for j in range(4):
            colbuf_ref[pl.ds(crow, 1), 4 + j:5 + j] = grow[:, j:j + 1]
        if is_fg:
            labv = jnp.where(v >= _FG_THRESH, grow[:, 4:5],
                             jnp.zeros((1, 1), f32))
            lab_acc = jnp.where((su_i == b) & (la_i == slot), labv, lab_acc)
        else:
            labv = jnp.zeros((1, 1), f32)
        colbuf_ref[pl.ds(crow, 1), 8:9] = labv
        return rm_b, lab_acc

    def body_fgbg(it, carry):
        rm_fg, rm_bg, lab_acc = carry
        rmf_parts, rmb_parts = [], []
        for b in range(nb):
            rm_b = rm_fg[b * 8:(b + 1) * 8, :]
            rm_b, lab_acc = step(b, rm_b, fg_ref, it, True, lab_acc)
            rmf_parts.append(rm_b)
            rm_b2 = rm_bg[b * 8:(b + 1) * 8, :]
            rm_b2, _ = step(b, rm_b2, bg_ref, _FG_ROIS + it, False, lab_acc)
            rmb_parts.append(rm_b2)
        return (jnp.concatenate(rmf_parts, axis=0),
                jnp.concatenate(rmb_parts, axis=0), lab_acc)

    def body_bg(it, carry):
        rm_bg, lab_acc = carry
        rmb_parts = []
        for b in range(nb):
            rm_b = rm_bg[b * 8:(b + 1) * 8, :]
            rm_b, _ = step(b, rm_b, bg_ref, _FG_ROIS + it, False, lab_acc)
            rmb_parts.append(rm_b)
        return jnp.concatenate(rmb_parts, axis=0), lab_acc

    lab0 = jnp.zeros((nb, _LANES), f32)
    rm_fg, rm_bg, lab_acc = lax.fori_loop(
        0, _FG_ROIS, body_fgbg, (rm_fg, rm_bg, lab0))
    rm_bg, lab_acc = lax.fori_loop(
        _FG_ROIS, _BG_ROIS, body_bg, (rm_bg, lab_acc))

    labels_ref[...] = lab_acc
    jj = lax.broadcasted_iota(i32, (_ROIS_PER_IMAGE, 4 * _NUM_CLASSES), 1)
    cj = jj >> 2
    dj = jj & 3
    for b in range(nb):
        cb = colbuf_ref[b * _ROIS_PER_IMAGE:(b + 1) * _ROIS_PER_IMAGE, :]
        ex1 = cb[:, 0:1]
        ey1 = cb[:, 1:2]
        ex2 = cb[:, 2:3]
        ey2 = cb[:, 3:4]
        gx1 = cb[:, 4:5]
        gy1 = cb[:, 5:6]
        gx2 = cb[:, 6:7]
        gy2 = cb[:, 7:8]
        cls = cb[:, 8:9]

        ew = ex2 - ex1 + 1.0
        eh = ey2 - ey1 + 1.0
        ecx = ex1 + 0.5 * ew
        ecy = ey1 + 0.5 * eh
        gw = gx2 - gx1 + 1.0
        gh = gy2 - gy1 + 1.0
        gcx = gx1 + 0.5 * gw
        gcy = gy1 + 0.5 * gh
        tx = ((gcx - ecx) / ew) / 0.1
        ty = ((gcy - ecy) / eh) / 0.1
        tw = jnp.log(gw / ew) / 0.2
        th = jnp.log(gh / eh) / 0.2

        bcol = jnp.full((_ROIS_PER_IMAGE, 1), float(b), f32)
        rois_ref[b] = jnp.concatenate([bcol, ex1, ey1, ex2, ey2], axis=1)
        clsi = cls.astype(i32)
        maskc = cls > 0.0
        sel = (cj == clsi) & maskc
        tval = jnp.where(dj == 0, tx,
                         jnp.where(dj == 1, ty, jnp.where(dj == 2, tw, th)))
        bbox_ref[b] = jnp.where(sel, tval, 0.0)
        inw_ref[b] = jnp.where(sel, 1.0, 0.0)


@jax.jit
def kernel(all_rois, gt_boxes, num_boxes):
    del num_boxes
    B, N, _ = all_rois.shape
    K = gt_boxes.shape[1]
    nk = N + K
    rows = -(-nk // _LANES)
    rows = -(-rows // 8) * 8
    p = rows * _LANES

    coords = jnp.concatenate([all_rois[:, :, 1:5], gt_boxes[:, :, :4]], axis=1)
    coords = jnp.pad(coords, ((0, 0), (0, p - nk), (0, 0)))
    planes = coords.transpose(0, 2, 1).reshape(B, 4, rows, _LANES)

    s1 = functools.partial(_stage1_kernel, nk=nk, rows=rows, k_gt=K)
    fg_s, bg_s, asg = pl.pallas_call(
        s1,
        grid=(B,),
        in_specs=[
            pl.BlockSpec((1, 4, rows, _LANES), lambda b: (b, 0, 0, 0)),
            pl.BlockSpec((1, K, 5), lambda b: (b, 0, 0)),
        ],
        out_specs=[
            pl.BlockSpec((1, rows, _LANES), lambda b: (b, 0, 0)),
            pl.BlockSpec((1, rows, _LANES), lambda b: (b, 0, 0)),
            pl.BlockSpec((1, rows, _LANES), lambda b: (b, 0, 0)),
        ],
        out_shape=(
            jax.ShapeDtypeStruct((B, rows, _LANES), jnp.float32),
            jax.ShapeDtypeStruct((B, rows, _LANES), jnp.float32),
            jax.ShapeDtypeStruct((B, rows, _LANES), jnp.int32),
        ),
    )(planes, gt_boxes)

    sel = functools.partial(_select_kernel, nb=B, rows=rows)
    full = lambda *shape: pl.BlockSpec(shape, lambda: tuple(0 for _ in shape))
    rois, labels, bbox, inw = pl.pallas_call(
        sel,
        grid=(),
        in_specs=[
            full(B, 4, rows, _LANES),
            full(B, K, 5),
            full(B, rows, _LANES),
            full(B, rows, _LANES),
            full(B, rows, _LANES),
        ],
        out_specs=[
            full(B, _ROIS_PER_IMAGE, 5),
            full(B, _ROIS_PER_IMAGE),
            full(B, _ROIS_PER_IMAGE, 4 * _NUM_CLASSES),
            full(B, _ROIS_PER_IMAGE, 4 * _NUM_CLASSES),
        ],
        out_shape=(
            jax.ShapeDtypeStruct((B, _ROIS_PER_IMAGE, 5), jnp.float32),
            jax.ShapeDtypeStruct((B, _ROIS_PER_IMAGE), jnp.float32),
            jax.ShapeDtypeStruct((B, _ROIS_PER_IMAGE, 4 * _NUM_CLASSES),
                                 jnp.float32),
            jax.ShapeDtypeStruct((B, _ROIS_PER_IMAGE, 4 * _NUM_CLASSES),
                                 jnp.float32),
        ),
        scratch_shapes=[
            pltpu.VMEM((B * rows, _LANES), jnp.float32),
            pltpu.VMEM((B * rows, _LANES), jnp.float32),
            pltpu.VMEM((B * _ROIS_PER_IMAGE, 16), jnp.float32),
        ],
    )(planes, gt_boxes, fg_s, bg_s, asg)
    return rois, labels, bbox, inw
